# Initial kernel scaffold; baseline (speedup 1.0000x reference)
import jax
import jax.numpy as jnp
from jax.experimental import pallas as pl


def kernel(z, edge_index, pool_edge2, pool_edge1, pool_edge0, W1, b1, W2, b2, W3, b3, W4, b4):
    raise NotImplementedError("write your pallas kernel here")


# speedup vs baseline: 4.9805x; 4.9805x over previous
import functools

import jax
import jax.numpy as jnp
from jax import lax
from jax.experimental import pallas as pl
from jax.experimental.pallas import tpu as pltpu
from jax.experimental.pallas import tpu_sc as plsc

F32 = jnp.float32
I32 = jnp.int32

NC = 2
NS = 16
BLK = 128


def _rows128(n):
    return ((n + 127) // 128) * 128


def _sc_mesh():
    return plsc.VectorSubcoreMesh(core_axis_name="c", subcore_axis_name="s")


def _make_count_kernel(specs):
    RA = _rows128(max(specs[0][0], specs[1][0]) + 1)
    RB = _rows128(max(specs[2][0], specs[3][0]) + 1)

    @functools.partial(
        pl.kernel,
        mesh=_sc_mesh(),
        compiler_params=pltpu.CompilerParams(use_tc_tiling_on_sc=False),
        out_type=tuple(
            jax.ShapeDtypeStruct((_rows128(M + 1), 16), F32)
            for (M, _) in specs
        ),
        scratch_types=[
            pltpu.VMEM_SHARED((RA, 16), F32),
            pltpu.VMEM_SHARED((RB, 16), F32),
            pltpu.VMEM((BLK, 16), F32),
            pltpu.VMEM((512, 16), F32),
            pltpu.VMEM((BLK,), I32),
            pltpu.VMEM((1, BLK), I32),
        ],
    )
    def count_k(d0, d1, d2, d3, o0, o1, o2, o3, cnt_a, cnt_b, ones, zb,
                dstv, idx):
        dsts = (d0, d1, d2, d3)
        outs = (o0, o1, o2, o3)
        c = lax.axis_index("c")
        s = lax.axis_index("s")

        def fill_ones(i, _):
            ones[i, :] = jnp.ones((16,), F32)
            return 0

        lax.fori_loop(0, BLK, fill_ones, 0)

        def fill_zeros(i, _):
            zb[i, :] = jnp.zeros((16,), F32)
            return 0

        lax.fori_loop(0, 512, fill_zeros, 0)

        for sh, rows in ((cnt_a, RA), (cnt_b, RB)):
            per = rows // NS
            off = 0
            while off < per:
                n = min(512, per - off)
                pltpu.sync_copy(zb.at[pl.ds(0, n)],
                                sh.at[pl.ds(s * per + off, n)])
                off += n
        plsc.subcore_barrier()

        for l, (M, Epad) in enumerate(specs):
            nblk = Epad // NS // BLK
            sh = cnt_a if l < 2 else cnt_b

            @pl.when(c == l % 2)
            def _(sh=sh, M=M, nblk=nblk, dst_hbm=dsts[l]):
                def blk_fn(b, _):
                    eoff = (s * nblk + b) * BLK
                    pltpu.sync_copy(dst_hbm.at[pl.ds(eoff, BLK)], dstv)
                    for g in range(BLK // 16):
                        d = dstv[pl.ds(g * 16, 16)]
                        ok = (d >= 0) & (d < M)
                        idx[0, pl.ds(g * 16, 16)] = jnp.where(ok, d, M)
                    pltpu.sync_copy(ones, sh.at[idx.at[0]], add=True)
                    return 0

                lax.fori_loop(0, nblk, blk_fn, 0)

        plsc.subcore_barrier()

        for l, (M, _) in enumerate(specs):
            per = _rows128(M + 1) // NS
            sh = cnt_a if l < 2 else cnt_b

            @pl.when(c == l % 2)
            def _(sh=sh, per=per, out=outs[l]):
                pltpu.sync_copy(sh.at[pl.ds(s * per, per)],
                                out.at[pl.ds(s * per, per)])

    return count_k


def _make_edge_kernel(M, Epad, csz, npass):
    assert 2 * npass * csz == M
    nblk = Epad // NS // BLK
    acc_rows = _rows128(csz + 1)
    zrows = acc_rows // NS
    wb = (csz // NS) & ~7
    wrem = csz - wb * NS

    @functools.partial(
        pl.kernel,
        mesh=_sc_mesh(),
        out_type=jax.ShapeDtypeStruct((M, 128), F32),
        scratch_types=[
            pltpu.VMEM_SHARED((acc_rows, 128), F32),
            pltpu.VMEM((BLK,), I32),
            pltpu.VMEM((BLK,), I32),
            pltpu.VMEM((1, BLK), I32),
            pltpu.VMEM((BLK, 128), F32),
            pltpu.SemaphoreType.DMA,
        ],
    )
    def edge_k(u_hbm, src_hbm, dst_hbm, acc_hbm, acc_sh, srcv, dstv, ldst,
               rows, sem):
        c = lax.axis_index("c")
        s = lax.axis_index("s")

        for p in range(npass):
            lo = (c * npass + p) * csz

            def zrow(i, _):
                rows[lax.div(i, 8), pl.ds(lax.rem(i, 8) * 16, 16)] = (
                    jnp.zeros((16,), F32))
                return 0

            lax.fori_loop(0, BLK * 8, zrow, 0)
            off = 0
            while off < zrows:
                n = min(BLK, zrows - off)
                pltpu.sync_copy(rows.at[pl.ds(0, n)],
                                acc_sh.at[pl.ds(s * zrows + off, n)])
                off += n
            plsc.subcore_barrier()

            def blk_fn(b, _):
                eoff = (s * nblk + b) * BLK
                pltpu.sync_copy(src_hbm.at[pl.ds(eoff, BLK)], srcv)
                pltpu.sync_copy(dst_hbm.at[pl.ds(eoff, BLK)], dstv)
                for g in range(BLK // 16):
                    d = dstv[pl.ds(g * 16, 16)]
                    ld = d - lo
                    ok = (ld >= 0) & (ld < csz)
                    ldst[0, pl.ds(g * 16, 16)] = jnp.where(ok, ld, csz)
                pltpu.async_copy(u_hbm.at[srcv], rows, sem).wait()
                pltpu.sync_copy(rows, acc_sh.at[ldst.at[0]], add=True)
                return 0

            lax.fori_loop(0, nblk, blk_fn, 0)
            plsc.subcore_barrier()

            pltpu.sync_copy(acc_sh.at[pl.ds(s * wb, wb)],
                            acc_hbm.at[pl.ds(lo + s * wb, wb)])
            if wrem:
                @pl.when(s == 0)
                def _():
                    pltpu.sync_copy(acc_sh.at[pl.ds(NS * wb, wrem)],
                                    acc_hbm.at[pl.ds(lo + NS * wb, wrem)])
            plsc.subcore_barrier()

    return edge_k


def _mm_first(z, w, cnt):
    mp, d = z.shape
    r = 1000

    def body(z_ref, w_ref, c_ref, o_ref):
        dis = lax.rsqrt(1.0 + c_ref[...][:, 0:1])
        o_ref[...] = jnp.dot(z_ref[...], w_ref[...],
                             preferred_element_type=F32) * dis

    return pl.pallas_call(
        body,
        grid=(mp // r,),
        in_specs=[
            pl.BlockSpec((r, d), lambda i: (i, 0)),
            pl.BlockSpec((d, d), lambda i: (0, 0)),
            pl.BlockSpec((r, 16), lambda i: (i, 0)),
        ],
        out_specs=pl.BlockSpec((r, d), lambda i: (i, 0)),
        out_shape=jax.ShapeDtypeStruct((mp, d), F32),
    )(z, w, cnt)


def _mm_mid(acc, u, cntp, bprev, w, cnt32):
    mp, d = acc.shape
    r = 1000

    def body(a_ref, u_ref, cp_ref, b_ref, w_ref, c32_ref, o_ref):
        disp = lax.rsqrt(1.0 + cp_ref[...][:, 0:1])
        x = jnp.maximum(disp * (a_ref[...] + u_ref[...]) + b_ref[...], 0.0)
        y = jnp.dot(x, w_ref[...], preferred_element_type=F32)
        c32 = c32_ref[...]
        d0 = lax.rsqrt(1.0 + c32[:, 0:1])
        d1 = lax.rsqrt(1.0 + c32[:, 16:17])
        o_ref[...] = jnp.concatenate([y * d0, y * d1], axis=1)

    return pl.pallas_call(
        body,
        grid=(mp // r,),
        in_specs=[
            pl.BlockSpec((r, d), lambda i: (i, 0)),
            pl.BlockSpec((r, d), lambda i: (i, 0)),
            pl.BlockSpec((r, 16), lambda i: (i, 0)),
            pl.BlockSpec((1, d), lambda i: (0, 0)),
            pl.BlockSpec((d, d), lambda i: (0, 0)),
            pl.BlockSpec((r, 32), lambda i: (i, 0)),
        ],
        out_specs=pl.BlockSpec((r, 2 * d), lambda i: (i, 0)),
        out_shape=jax.ShapeDtypeStruct((mp, 2 * d), F32),
    )(acc, u, cntp, bprev, w, cnt32)


def _fin_last(acc, u, cnt, b):
    m, d = acc.shape
    r = 2000

    def body(a_ref, u_ref, c_ref, b_ref, o_ref):
        dis = lax.rsqrt(1.0 + c_ref[...][:, 0:1])
        o_ref[...] = dis * (a_ref[...] + u_ref[...]) + b_ref[...]

    return pl.pallas_call(
        body,
        grid=(m // r,),
        in_specs=[
            pl.BlockSpec((r, d), lambda i: (i, 0)),
            pl.BlockSpec((r, d), lambda i: (i, 0)),
            pl.BlockSpec((r, 16), lambda i: (i, 0)),
            pl.BlockSpec((1, d), lambda i: (0, 0)),
        ],
        out_specs=pl.BlockSpec((r, d), lambda i: (i, 0)),
        out_shape=jax.ShapeDtypeStruct((m, d), F32),
    )(acc, u, cnt, b)


def _pad_edges(e):
    n = e.shape[1]
    p = (-n) % (NS * BLK)
    src = jnp.concatenate([e[0].astype(I32), jnp.zeros((p,), I32)])
    dst = jnp.concatenate([e[1].astype(I32), jnp.full((p,), -1, I32)])
    return src, dst


def kernel(z, edge_index, pool_edge2, pool_edge1, pool_edge0,
           W1, b1, W2, b2, W3, b3, W4, b4):
    n = z.shape[0]
    s1, d1 = _pad_edges(edge_index)
    s2, d2 = _pad_edges(pool_edge2)
    s3, d3 = _pad_edges(pool_edge1)
    s4, d4 = _pad_edges(pool_edge0)

    specs = [(n, s1.shape[0]), (2 * n, s2.shape[0]),
             (4 * n, s3.shape[0]), (8 * n, s4.shape[0])]
    cnt1, cnt2, cnt3, cnt4 = _make_count_kernel(specs)(d1, d2, d3, d4)

    u1 = _mm_first(z, W1, cnt1)
    acc1 = _make_edge_kernel(n, s1.shape[0], n // 2, 1)(u1, s1, d1)

    u2 = _mm_mid(acc1, u1, cnt1, b1.reshape(1, -1), W2,
                 cnt2.reshape(-1, 32)).reshape(2 * n, -1)
    acc2 = _make_edge_kernel(2 * n, s2.shape[0], n, 1)(u2, s2, d2)

    u3 = _mm_mid(acc2, u2, cnt2, b2.reshape(1, -1), W3,
                 cnt3.reshape(-1, 32)).reshape(4 * n, -1)
    acc3 = _make_edge_kernel(4 * n, s3.shape[0], n, 2)(u3, s3, d3)

    u4 = _mm_mid(acc3, u3, cnt3, b3.reshape(1, -1), W4,
                 cnt4.reshape(-1, 32)).reshape(8 * n, -1)
    acc4 = _make_edge_kernel(8 * n, s4.shape[0], n, 4)(u4, s4, d4)

    return _fin_last(acc4, u4, cnt4, b4.reshape(1, -1))


# speedup vs baseline: 8.6977x; 1.7464x over previous
import functools

import jax
import jax.numpy as jnp
from jax import lax
from jax.experimental import pallas as pl
from jax.experimental.pallas import tpu as pltpu
from jax.experimental.pallas import tpu_sc as plsc

F32 = jnp.float32
I32 = jnp.int32

NC = 2
NS = 16
BLK = 128


def _rows128(n):
    return ((n + 127) // 128) * 128


def _sc_mesh():
    return plsc.VectorSubcoreMesh(core_axis_name="c", subcore_axis_name="s")


def _make_count_kernel(specs):
    RA = _rows128(max(specs[0][0], specs[1][0]) + 1)
    RB = _rows128(max(specs[2][0], specs[3][0]) + 1)

    @functools.partial(
        pl.kernel,
        mesh=_sc_mesh(),
        compiler_params=pltpu.CompilerParams(use_tc_tiling_on_sc=False),
        out_type=tuple(
            jax.ShapeDtypeStruct((_rows128(M + 1), 16), F32)
            for (M, _) in specs
        ),
        scratch_types=[
            pltpu.VMEM_SHARED((RA, 16), F32),
            pltpu.VMEM_SHARED((RB, 16), F32),
            pltpu.VMEM((BLK, 16), F32),
            pltpu.VMEM((512, 16), F32),
            pltpu.VMEM((BLK,), I32),
            pltpu.VMEM((1, BLK), I32),
        ],
    )
    def count_k(d0, d1, d2, d3, o0, o1, o2, o3, cnt_a, cnt_b, ones, zb,
                dstv, idx):
        dsts = (d0, d1, d2, d3)
        outs = (o0, o1, o2, o3)
        c = lax.axis_index("c")
        s = lax.axis_index("s")

        def fill_ones(i, _):
            ones[i, :] = jnp.ones((16,), F32)
            return 0

        lax.fori_loop(0, BLK, fill_ones, 0)

        def fill_zeros(i, _):
            zb[i, :] = jnp.zeros((16,), F32)
            return 0

        lax.fori_loop(0, 512, fill_zeros, 0)

        for sh, rows in ((cnt_a, RA), (cnt_b, RB)):
            per = rows // NS
            off = 0
            while off < per:
                n = min(512, per - off)
                pltpu.sync_copy(zb.at[pl.ds(0, n)],
                                sh.at[pl.ds(s * per + off, n)])
                off += n
        plsc.subcore_barrier()

        for l, (M, Epad) in enumerate(specs):
            nblk = Epad // NS // BLK
            sh = cnt_a if l < 2 else cnt_b

            @pl.when(c == l % 2)
            def _(sh=sh, M=M, nblk=nblk, dst_hbm=dsts[l]):
                def blk_fn(b, _):
                    eoff = (s * nblk + b) * BLK
                    pltpu.sync_copy(dst_hbm.at[pl.ds(eoff, BLK)], dstv)
                    for g in range(BLK // 16):
                        d = dstv[pl.ds(g * 16, 16)]
                        ok = (d >= 0) & (d < M)
                        idx[0, pl.ds(g * 16, 16)] = jnp.where(ok, d, M)
                    pltpu.sync_copy(ones, sh.at[idx.at[0]], add=True)
                    return 0

                lax.fori_loop(0, nblk, blk_fn, 0)

        plsc.subcore_barrier()

        for l, (M, _) in enumerate(specs):
            per = _rows128(M + 1) // NS
            sh = cnt_a if l < 2 else cnt_b

            @pl.when(c == l % 2)
            def _(sh=sh, per=per, out=outs[l]):
                pltpu.sync_copy(sh.at[pl.ds(s * per, per)],
                                out.at[pl.ds(s * per, per)])

    return count_k


def _make_edge_kernel(M, Epad, csz, npass):
    assert 2 * npass * csz == M
    nblk = Epad // NS // BLK
    acc_rows = _rows128(csz + 1)
    zrows = acc_rows // NS
    wb = (csz // NS) & ~7
    wrem = csz - wb * NS

    @functools.partial(
        pl.kernel,
        mesh=_sc_mesh(),
        compiler_params=pltpu.CompilerParams(needs_layout_passes=False),
        out_type=jax.ShapeDtypeStruct((M, 128), F32),
        scratch_types=[
            pltpu.VMEM_SHARED((acc_rows, 128), F32),
            pltpu.VMEM((BLK,), I32),
            pltpu.VMEM((BLK,), I32),
            pltpu.VMEM((2, BLK), I32),
            pltpu.VMEM((2, BLK), I32),
            pltpu.VMEM((BLK, 128), F32),
            pltpu.SemaphoreType.DMA,
        ],
    )
    def edge_k(u_hbm, src_hbm, dst_hbm, acc_hbm, acc_sh, srcv, dstv, sring,
               lring, rows, sem):
        c = lax.axis_index("c")
        s = lax.axis_index("s")
        lanes = lax.iota(I32, 16)

        def flush(fl):
            parity = lax.rem(lax.div(fl, BLK), 2)
            for q in range(2):
                @pl.when(parity == q)
                def _(q=q):
                    pltpu.async_copy(u_hbm.at[sring.at[q]], rows,
                                     sem).wait()
                    pltpu.sync_copy(rows, acc_sh.at[lring.at[q]], add=True)

        for p in range(npass):
            lo = (c * npass + p) * csz

            def zrow(i, _):
                rows[lax.div(i, 8), pl.ds(lax.rem(i, 8) * 16, 16)] = (
                    jnp.zeros((16,), F32))
                return 0

            lax.fori_loop(0, BLK * 8, zrow, 0)
            off = 0
            while off < zrows:
                n = min(BLK, zrows - off)
                pltpu.sync_copy(rows.at[pl.ds(0, n)],
                                acc_sh.at[pl.ds(s * zrows + off, n)])
                off += n
            plsc.subcore_barrier()

            def blk_fn(b, carry):
                cnt, fl = carry
                eoff = (s * nblk + b) * BLK
                pltpu.sync_copy(src_hbm.at[pl.ds(eoff, BLK)], srcv)
                pltpu.sync_copy(dst_hbm.at[pl.ds(eoff, BLK)], dstv)
                for g in range(BLK // 16):
                    d = dstv[pl.ds(g * 16, 16)]
                    sv = srcv[pl.ds(g * 16, 16)]
                    ld = d - lo
                    ok = (ld >= 0) & (ld < csz)
                    pos = cnt + jnp.cumsum(ok.astype(I32)) - 1
                    slot = pos & (2 * BLK - 1)
                    srow = slot >> 7
                    scol = slot & (BLK - 1)
                    plsc.store_scatter(sring, [srow, scol], sv, mask=ok)
                    plsc.store_scatter(lring, [srow, scol], ld, mask=ok)
                    cnt = cnt + jnp.sum(ok.astype(I32))

                @pl.when(cnt - fl >= BLK)
                def _():
                    flush(fl)

                fl = jnp.where(cnt - fl >= BLK, fl + BLK, fl)
                return cnt, fl

            cnt, fl = lax.fori_loop(0, nblk, blk_fn,
                                    (jnp.int32(0), jnp.int32(0)))

            for g in range(BLK // 16):
                slot = (cnt + g * 16 + lanes) & (2 * BLK - 1)
                srow = slot >> 7
                scol = slot & (BLK - 1)
                plsc.store_scatter(sring, [srow, scol],
                                   jnp.zeros((16,), I32))
                plsc.store_scatter(lring, [srow, scol],
                                   jnp.full((16,), csz, I32))
            flush(fl)
            plsc.subcore_barrier()

            pltpu.sync_copy(acc_sh.at[pl.ds(s * wb, wb)],
                            acc_hbm.at[pl.ds(lo + s * wb, wb)])
            if wrem:
                @pl.when(s == 0)
                def _():
                    pltpu.sync_copy(acc_sh.at[pl.ds(NS * wb, wrem)],
                                    acc_hbm.at[pl.ds(lo + NS * wb, wrem)])
            plsc.subcore_barrier()

    return edge_k


def _mm_first(z, w, cnt):
    mp, d = z.shape
    r = 1000

    def body(z_ref, w_ref, c_ref, o_ref):
        dis = lax.rsqrt(1.0 + c_ref[...][:, 0:1])
        o_ref[...] = jnp.dot(z_ref[...], w_ref[...],
                             preferred_element_type=F32) * dis

    return pl.pallas_call(
        body,
        grid=(mp // r,),
        in_specs=[
            pl.BlockSpec((r, d), lambda i: (i, 0)),
            pl.BlockSpec((d, d), lambda i: (0, 0)),
            pl.BlockSpec((r, 16), lambda i: (i, 0)),
        ],
        out_specs=pl.BlockSpec((r, d), lambda i: (i, 0)),
        out_shape=jax.ShapeDtypeStruct((mp, d), F32),
    )(z, w, cnt)


def _mm_mid(acc, u, cntp, bprev, w, cnt32):
    mp, d = acc.shape
    r = 1000

    def body(a_ref, u_ref, cp_ref, b_ref, w_ref, c32_ref, o_ref):
        disp = lax.rsqrt(1.0 + cp_ref[...][:, 0:1])
        x = jnp.maximum(disp * (a_ref[...] + u_ref[...]) + b_ref[...], 0.0)
        y = jnp.dot(x, w_ref[...], preferred_element_type=F32)
        c32 = c32_ref[...]
        d0 = lax.rsqrt(1.0 + c32[:, 0:1])
        d1 = lax.rsqrt(1.0 + c32[:, 16:17])
        o_ref[...] = jnp.concatenate([y * d0, y * d1], axis=1)

    return pl.pallas_call(
        body,
        grid=(mp // r,),
        in_specs=[
            pl.BlockSpec((r, d), lambda i: (i, 0)),
            pl.BlockSpec((r, d), lambda i: (i, 0)),
            pl.BlockSpec((r, 16), lambda i: (i, 0)),
            pl.BlockSpec((1, d), lambda i: (0, 0)),
            pl.BlockSpec((d, d), lambda i: (0, 0)),
            pl.BlockSpec((r, 32), lambda i: (i, 0)),
        ],
        out_specs=pl.BlockSpec((r, 2 * d), lambda i: (i, 0)),
        out_shape=jax.ShapeDtypeStruct((mp, 2 * d), F32),
    )(acc, u, cntp, bprev, w, cnt32)


def _fin_last(acc, u, cnt, b):
    m, d = acc.shape
    r = 2000

    def body(a_ref, u_ref, c_ref, b_ref, o_ref):
        dis = lax.rsqrt(1.0 + c_ref[...][:, 0:1])
        o_ref[...] = dis * (a_ref[...] + u_ref[...]) + b_ref[...]

    return pl.pallas_call(
        body,
        grid=(m // r,),
        in_specs=[
            pl.BlockSpec((r, d), lambda i: (i, 0)),
            pl.BlockSpec((r, d), lambda i: (i, 0)),
            pl.BlockSpec((r, 16), lambda i: (i, 0)),
            pl.BlockSpec((1, d), lambda i: (0, 0)),
        ],
        out_specs=pl.BlockSpec((r, d), lambda i: (i, 0)),
        out_shape=jax.ShapeDtypeStruct((m, d), F32),
    )(acc, u, cnt, b)


def _pad_edges(e):
    n = e.shape[1]
    p = (-n) % (NS * BLK)
    src = jnp.concatenate([e[0].astype(I32), jnp.zeros((p,), I32)])
    dst = jnp.concatenate([e[1].astype(I32), jnp.full((p,), -1, I32)])
    return src, dst


def kernel(z, edge_index, pool_edge2, pool_edge1, pool_edge0,
           W1, b1, W2, b2, W3, b3, W4, b4):
    n = z.shape[0]
    s1, d1 = _pad_edges(edge_index)
    s2, d2 = _pad_edges(pool_edge2)
    s3, d3 = _pad_edges(pool_edge1)
    s4, d4 = _pad_edges(pool_edge0)

    specs = [(n, s1.shape[0]), (2 * n, s2.shape[0]),
             (4 * n, s3.shape[0]), (8 * n, s4.shape[0])]
    cnt1, cnt2, cnt3, cnt4 = _make_count_kernel(specs)(d1, d2, d3, d4)

    u1 = _mm_first(z, W1, cnt1)
    acc1 = _make_edge_kernel(n, s1.shape[0], n // 2, 1)(u1, s1, d1)

    u2 = _mm_mid(acc1, u1, cnt1, b1.reshape(1, -1), W2,
                 cnt2.reshape(-1, 32)).reshape(2 * n, -1)
    acc2 = _make_edge_kernel(2 * n, s2.shape[0], n, 1)(u2, s2, d2)

    u3 = _mm_mid(acc2, u2, cnt2, b2.reshape(1, -1), W3,
                 cnt3.reshape(-1, 32)).reshape(4 * n, -1)
    acc3 = _make_edge_kernel(4 * n, s3.shape[0], n, 2)(u3, s3, d3)

    u4 = _mm_mid(acc3, u3, cnt3, b3.reshape(1, -1), W4,
                 cnt4.reshape(-1, 32)).reshape(8 * n, -1)
    acc4 = _make_edge_kernel(8 * n, s4.shape[0], n, 4)(u4, s4, d4)

    return _fin_last(acc4, u4, cnt4, b4.reshape(1, -1))


# speedup vs baseline: 12.3441x; 1.4192x over previous
import functools

import jax
import jax.numpy as jnp
from jax import lax
from jax.experimental import pallas as pl
from jax.experimental.pallas import tpu as pltpu
from jax.experimental.pallas import tpu_sc as plsc

F32 = jnp.float32
I32 = jnp.int32

NC = 2
NS = 16
BLK = 128


def _rows128(n):
    return ((n + 127) // 128) * 128


def _sc_mesh():
    return plsc.VectorSubcoreMesh(core_axis_name="c", subcore_axis_name="s")


def _make_count_kernel(specs):
    RA = _rows128(max(specs[0][0], specs[1][0]) + 1)
    RB = _rows128(max(specs[2][0], specs[3][0]) + 1)

    @functools.partial(
        pl.kernel,
        mesh=_sc_mesh(),
        compiler_params=pltpu.CompilerParams(use_tc_tiling_on_sc=False),
        out_type=tuple(
            jax.ShapeDtypeStruct((_rows128(M + 1), 16), F32)
            for (M, _) in specs
        ),
        scratch_types=[
            pltpu.VMEM_SHARED((RA, 16), F32),
            pltpu.VMEM_SHARED((RB, 16), F32),
            pltpu.VMEM((BLK, 16), F32),
            pltpu.VMEM((512, 16), F32),
            pltpu.VMEM((BLK,), I32),
            pltpu.VMEM((1, BLK), I32),
        ],
    )
    def count_k(d0, d1, d2, d3, o0, o1, o2, o3, cnt_a, cnt_b, ones, zb,
                dstv, idx):
        dsts = (d0, d1, d2, d3)
        outs = (o0, o1, o2, o3)
        c = lax.axis_index("c")
        s = lax.axis_index("s")

        def fill_ones(i, _):
            ones[i, :] = jnp.ones((16,), F32)
            return 0

        lax.fori_loop(0, BLK, fill_ones, 0)

        def fill_zeros(i, _):
            zb[i, :] = jnp.zeros((16,), F32)
            return 0

        lax.fori_loop(0, 512, fill_zeros, 0)

        for sh, rows in ((cnt_a, RA), (cnt_b, RB)):
            per = rows // NS
            off = 0
            while off < per:
                n = min(512, per - off)
                pltpu.sync_copy(zb.at[pl.ds(0, n)],
                                sh.at[pl.ds(s * per + off, n)])
                off += n
        plsc.subcore_barrier()

        for l, (M, Epad) in enumerate(specs):
            nblk = Epad // NS // BLK
            sh = cnt_a if l < 2 else cnt_b

            @pl.when(c == l % 2)
            def _(sh=sh, M=M, nblk=nblk, dst_hbm=dsts[l]):
                def blk_fn(b, _):
                    eoff = (s * nblk + b) * BLK
                    pltpu.sync_copy(dst_hbm.at[pl.ds(eoff, BLK)], dstv)
                    for g in range(BLK // 16):
                        d = dstv[pl.ds(g * 16, 16)]
                        ok = (d >= 0) & (d < M)
                        idx[0, pl.ds(g * 16, 16)] = jnp.where(ok, d, M)
                    pltpu.sync_copy(ones, sh.at[idx.at[0]], add=True)
                    return 0

                lax.fori_loop(0, nblk, blk_fn, 0)

        plsc.subcore_barrier()

        for l, (M, _) in enumerate(specs):
            per = _rows128(M + 1) // NS
            sh = cnt_a if l < 2 else cnt_b

            @pl.when(c == l % 2)
            def _(sh=sh, per=per, out=outs[l]):
                pltpu.sync_copy(sh.at[pl.ds(s * per, per)],
                                out.at[pl.ds(s * per, per)])

    return count_k


def _make_edge_kernel(M, Epad, csz, npass):
    assert 2 * npass * csz == M
    nblk = Epad // NS // BLK
    acc_rows = _rows128(csz + 1)
    zrows = acc_rows // NS
    wb = (csz // NS) & ~7
    wrem = csz - wb * NS

    @functools.partial(
        pl.kernel,
        mesh=_sc_mesh(),
        compiler_params=pltpu.CompilerParams(needs_layout_passes=False),
        out_type=jax.ShapeDtypeStruct((M, 128), F32),
        scratch_types=[
            pltpu.VMEM_SHARED((acc_rows, 128), F32),
            pltpu.VMEM((2, BLK), I32),
            pltpu.VMEM((2, BLK), I32),
            pltpu.VMEM((2, BLK), I32),
            pltpu.VMEM((2, BLK), I32),
            pltpu.VMEM((BLK, 128), F32),
            pltpu.SemaphoreType.DMA,
            pltpu.SemaphoreType.DMA,
        ],
    )
    def edge_k(u_hbm, src_hbm, dst_hbm, acc_hbm, acc_sh, srcv, dstv, sring,
               lring, rows, sem, sem_idx):
        c = lax.axis_index("c")
        s = lax.axis_index("s")
        lanes = lax.iota(I32, 16)

        def issue(b):
            q = b & 1
            eoff = (s * nblk + b) * BLK
            pltpu.async_copy(src_hbm.at[pl.ds(eoff, BLK)], srcv.at[q],
                             sem_idx)
            pltpu.async_copy(dst_hbm.at[pl.ds(eoff, BLK)], dstv.at[q],
                             sem_idx)

        def drain(b):
            q = b & 1
            eoff = (s * nblk + b) * BLK
            pltpu.make_async_copy(src_hbm.at[pl.ds(eoff, BLK)], srcv.at[q],
                                  sem_idx).wait()
            pltpu.make_async_copy(dst_hbm.at[pl.ds(eoff, BLK)], dstv.at[q],
                                  sem_idx).wait()

        def flush(fl):
            parity = lax.rem(lax.div(fl, BLK), 2)
            for q in range(2):
                @pl.when(parity == q)
                def _(q=q):
                    pltpu.async_copy(u_hbm.at[sring.at[q]], rows,
                                     sem).wait()
                    pltpu.sync_copy(rows, acc_sh.at[lring.at[q]], add=True)

        for p in range(npass):
            lo = (c * npass + p) * csz

            def zrow(i, _):
                rows[lax.div(i, 8), pl.ds(lax.rem(i, 8) * 16, 16)] = (
                    jnp.zeros((16,), F32))
                return 0

            lax.fori_loop(0, BLK * 8, zrow, 0)
            off = 0
            while off < zrows:
                n = min(BLK, zrows - off)
                pltpu.sync_copy(rows.at[pl.ds(0, n)],
                                acc_sh.at[pl.ds(s * zrows + off, n)])
                off += n
            plsc.subcore_barrier()

            def blk_fn(b, carry):
                cnt, fl = carry
                q = b & 1
                drain(b)

                @pl.when(b + 1 < nblk)
                def _():
                    issue(b + 1)

                for g in range(BLK // 16):
                    d = dstv[q, pl.ds(g * 16, 16)]
                    sv = srcv[q, pl.ds(g * 16, 16)]
                    ld = d - lo
                    ok = (ld >= 0) & (ld < csz)
                    pos = cnt + jnp.cumsum(ok.astype(I32)) - 1
                    slot = pos & (2 * BLK - 1)
                    srow = slot >> 7
                    scol = slot & (BLK - 1)
                    plsc.store_scatter(sring, [srow, scol], sv, mask=ok)
                    plsc.store_scatter(lring, [srow, scol], ld, mask=ok)
                    cnt = cnt + jnp.sum(ok.astype(I32))

                @pl.when(cnt - fl >= BLK)
                def _():
                    flush(fl)

                fl = jnp.where(cnt - fl >= BLK, fl + BLK, fl)
                return cnt, fl

            issue(jnp.int32(0))
            cnt, fl = lax.fori_loop(0, nblk, blk_fn,
                                    (jnp.int32(0), jnp.int32(0)))

            for g in range(BLK // 16):
                slot = (cnt + g * 16 + lanes) & (2 * BLK - 1)
                srow = slot >> 7
                scol = slot & (BLK - 1)
                plsc.store_scatter(sring, [srow, scol],
                                   jnp.zeros((16,), I32))
                plsc.store_scatter(lring, [srow, scol],
                                   jnp.full((16,), csz, I32))
            flush(fl)
            plsc.subcore_barrier()

            pltpu.sync_copy(acc_sh.at[pl.ds(s * wb, wb)],
                            acc_hbm.at[pl.ds(lo + s * wb, wb)])
            if wrem:
                @pl.when(s == 0)
                def _():
                    pltpu.sync_copy(acc_sh.at[pl.ds(NS * wb, wrem)],
                                    acc_hbm.at[pl.ds(lo + NS * wb, wrem)])
            plsc.subcore_barrier()

    return edge_k


def _mm_first(z, w, cnt):
    mp, d = z.shape
    r = 1000

    def body(z_ref, w_ref, c_ref, o_ref):
        dis = lax.rsqrt(1.0 + c_ref[...][:, 0:1])
        o_ref[...] = jnp.dot(z_ref[...], w_ref[...],
                             preferred_element_type=F32) * dis

    return pl.pallas_call(
        body,
        grid=(mp // r,),
        in_specs=[
            pl.BlockSpec((r, d), lambda i: (i, 0)),
            pl.BlockSpec((d, d), lambda i: (0, 0)),
            pl.BlockSpec((r, 16), lambda i: (i, 0)),
        ],
        out_specs=pl.BlockSpec((r, d), lambda i: (i, 0)),
        out_shape=jax.ShapeDtypeStruct((mp, d), F32),
    )(z, w, cnt)


def _mm_mid(acc, u, cntp, bprev, w, cnt32):
    mp, d = acc.shape
    r = 1000

    def body(a_ref, u_ref, cp_ref, b_ref, w_ref, c32_ref, o_ref):
        disp = lax.rsqrt(1.0 + cp_ref[...][:, 0:1])
        x = jnp.maximum(disp * (a_ref[...] + u_ref[...]) + b_ref[...], 0.0)
        y = jnp.dot(x, w_ref[...], preferred_element_type=F32)
        c32 = c32_ref[...]
        d0 = lax.rsqrt(1.0 + c32[:, 0:1])
        d1 = lax.rsqrt(1.0 + c32[:, 16:17])
        o_ref[...] = jnp.concatenate([y * d0, y * d1], axis=1)

    return pl.pallas_call(
        body,
        grid=(mp // r,),
        in_specs=[
            pl.BlockSpec((r, d), lambda i: (i, 0)),
            pl.BlockSpec((r, d), lambda i: (i, 0)),
            pl.BlockSpec((r, 16), lambda i: (i, 0)),
            pl.BlockSpec((1, d), lambda i: (0, 0)),
            pl.BlockSpec((d, d), lambda i: (0, 0)),
            pl.BlockSpec((r, 32), lambda i: (i, 0)),
        ],
        out_specs=pl.BlockSpec((r, 2 * d), lambda i: (i, 0)),
        out_shape=jax.ShapeDtypeStruct((mp, 2 * d), F32),
    )(acc, u, cntp, bprev, w, cnt32)


def _fin_last(acc, u, cnt, b):
    m, d = acc.shape
    r = 2000

    def body(a_ref, u_ref, c_ref, b_ref, o_ref):
        dis = lax.rsqrt(1.0 + c_ref[...][:, 0:1])
        o_ref[...] = dis * (a_ref[...] + u_ref[...]) + b_ref[...]

    return pl.pallas_call(
        body,
        grid=(m // r,),
        in_specs=[
            pl.BlockSpec((r, d), lambda i: (i, 0)),
            pl.BlockSpec((r, d), lambda i: (i, 0)),
            pl.BlockSpec((r, 16), lambda i: (i, 0)),
            pl.BlockSpec((1, d), lambda i: (0, 0)),
        ],
        out_specs=pl.BlockSpec((r, d), lambda i: (i, 0)),
        out_shape=jax.ShapeDtypeStruct((m, d), F32),
    )(acc, u, cnt, b)


def _pad_edges(e):
    n = e.shape[1]
    p = (-n) % (NS * BLK)
    src = jnp.concatenate([e[0].astype(I32), jnp.zeros((p,), I32)])
    dst = jnp.concatenate([e[1].astype(I32), jnp.full((p,), -1, I32)])
    return src, dst


def kernel(z, edge_index, pool_edge2, pool_edge1, pool_edge0,
           W1, b1, W2, b2, W3, b3, W4, b4):
    n = z.shape[0]
    s1, d1 = _pad_edges(edge_index)
    s2, d2 = _pad_edges(pool_edge2)
    s3, d3 = _pad_edges(pool_edge1)
    s4, d4 = _pad_edges(pool_edge0)

    specs = [(n, s1.shape[0]), (2 * n, s2.shape[0]),
             (4 * n, s3.shape[0]), (8 * n, s4.shape[0])]
    cnt1, cnt2, cnt3, cnt4 = _make_count_kernel(specs)(d1, d2, d3, d4)

    u1 = _mm_first(z, W1, cnt1)
    acc1 = _make_edge_kernel(n, s1.shape[0], n // 2, 1)(u1, s1, d1)

    u2 = _mm_mid(acc1, u1, cnt1, b1.reshape(1, -1), W2,
                 cnt2.reshape(-1, 32)).reshape(2 * n, -1)
    acc2 = _make_edge_kernel(2 * n, s2.shape[0], n, 1)(u2, s2, d2)

    u3 = _mm_mid(acc2, u2, cnt2, b2.reshape(1, -1), W3,
                 cnt3.reshape(-1, 32)).reshape(4 * n, -1)
    acc3 = _make_edge_kernel(4 * n, s3.shape[0], n, 2)(u3, s3, d3)

    u4 = _mm_mid(acc3, u3, cnt3, b3.reshape(1, -1), W4,
                 cnt4.reshape(-1, 32)).reshape(8 * n, -1)
    acc4 = _make_edge_kernel(8 * n, s4.shape[0], n, 4)(u4, s4, d4)

    return _fin_last(acc4, u4, cnt4, b4.reshape(1, -1))


# speedup vs baseline: 12.6297x; 1.0231x over previous
import functools

import jax
import jax.numpy as jnp
from jax import lax
from jax.experimental import pallas as pl
from jax.experimental.pallas import tpu as pltpu
from jax.experimental.pallas import tpu_sc as plsc

F32 = jnp.float32
I32 = jnp.int32

NC = 2
NS = 16
BLK = 128


def _rows128(n):
    return ((n + 127) // 128) * 128


def _sc_mesh():
    return plsc.VectorSubcoreMesh(core_axis_name="c", subcore_axis_name="s")


def _make_count_kernel(specs):
    RA = _rows128(max(specs[0][0], specs[1][0]) + 1)
    RB = _rows128(max(specs[2][0], specs[3][0]) + 1)

    @functools.partial(
        pl.kernel,
        mesh=_sc_mesh(),
        compiler_params=pltpu.CompilerParams(use_tc_tiling_on_sc=False,
                                             needs_layout_passes=False),
        out_type=tuple(
            jax.ShapeDtypeStruct((_rows128(M + 1), 16), F32)
            for (M, _) in specs
        ),
        scratch_types=[
            pltpu.VMEM_SHARED((RA, 16), F32),
            pltpu.VMEM_SHARED((RB, 16), F32),
            pltpu.VMEM((BLK, 16), F32),
            pltpu.VMEM((512, 16), F32),
            pltpu.VMEM((2, BLK), I32),
            pltpu.VMEM((1, BLK), I32),
            pltpu.SemaphoreType.DMA,
        ],
    )
    def count_k(d0, d1, d2, d3, o0, o1, o2, o3, cnt_a, cnt_b, ones, zb,
                dstv, idx, sem_idx):
        dsts = (d0, d1, d2, d3)
        outs = (o0, o1, o2, o3)
        c = lax.axis_index("c")
        s = lax.axis_index("s")

        def fill_ones(i, _):
            ones[i, :] = jnp.ones((16,), F32)
            return 0

        lax.fori_loop(0, BLK, fill_ones, 0)

        def fill_zeros(i, _):
            zb[i, :] = jnp.zeros((16,), F32)
            return 0

        lax.fori_loop(0, 512, fill_zeros, 0)

        for sh, rows in ((cnt_a, RA), (cnt_b, RB)):
            per = rows // NS
            off = 0
            while off < per:
                n = min(512, per - off)
                pltpu.sync_copy(zb.at[pl.ds(0, n)],
                                sh.at[pl.ds(s * per + off, n)])
                off += n
        plsc.subcore_barrier()

        for l, (M, Epad) in enumerate(specs):
            nblk = Epad // NS // BLK
            sh = cnt_a if l < 2 else cnt_b

            @pl.when(c == l % 2)
            def _(sh=sh, M=M, nblk=nblk, dst_hbm=dsts[l]):
                def issue(b):
                    eoff = (s * nblk + b) * BLK
                    pltpu.async_copy(dst_hbm.at[pl.ds(eoff, BLK)],
                                     dstv.at[b & 1], sem_idx)

                def blk_fn(b, _):
                    eoff = (s * nblk + b) * BLK
                    q = b & 1
                    pltpu.make_async_copy(dst_hbm.at[pl.ds(eoff, BLK)],
                                          dstv.at[q], sem_idx).wait()

                    @pl.when(b + 1 < nblk)
                    def _():
                        issue(b + 1)

                    for g in range(BLK // 16):
                        d = dstv[q, pl.ds(g * 16, 16)]
                        ok = (d >= 0) & (d < M)
                        idx[0, pl.ds(g * 16, 16)] = jnp.where(ok, d, M)
                    pltpu.sync_copy(ones, sh.at[idx.at[0]], add=True)
                    return 0

                issue(jnp.int32(0))
                lax.fori_loop(0, nblk, blk_fn, 0)

        plsc.subcore_barrier()

        for l, (M, _) in enumerate(specs):
            per = _rows128(M + 1) // NS
            sh = cnt_a if l < 2 else cnt_b

            @pl.when(c == l % 2)
            def _(sh=sh, per=per, out=outs[l]):
                pltpu.sync_copy(sh.at[pl.ds(s * per, per)],
                                out.at[pl.ds(s * per, per)])

    return count_k


def _make_edge_kernel(M, Epad, csz, npass):
    assert 2 * npass * csz == M
    nblk = Epad // NS // BLK
    acc_rows = _rows128(csz + 1)
    zrows = acc_rows // NS
    wb = (csz // NS) & ~7
    wrem = csz - wb * NS

    @functools.partial(
        pl.kernel,
        mesh=_sc_mesh(),
        compiler_params=pltpu.CompilerParams(needs_layout_passes=False),
        out_type=jax.ShapeDtypeStruct((M, 128), F32),
        scratch_types=[
            pltpu.VMEM_SHARED((acc_rows, 128), F32),
            pltpu.VMEM((2, BLK), I32),
            pltpu.VMEM((2, BLK), I32),
            pltpu.VMEM((2, BLK), I32),
            pltpu.VMEM((2, BLK), I32),
            pltpu.VMEM((BLK, 128), F32),
            pltpu.SemaphoreType.DMA,
            pltpu.SemaphoreType.DMA,
        ],
    )
    def edge_k(u_hbm, src_hbm, dst_hbm, acc_hbm, acc_sh, srcv, dstv, sring,
               lring, rows, sem, sem_idx):
        c = lax.axis_index("c")
        s = lax.axis_index("s")
        lanes = lax.iota(I32, 16)

        def issue(b):
            q = b & 1
            eoff = (s * nblk + b) * BLK
            pltpu.async_copy(src_hbm.at[pl.ds(eoff, BLK)], srcv.at[q],
                             sem_idx)
            pltpu.async_copy(dst_hbm.at[pl.ds(eoff, BLK)], dstv.at[q],
                             sem_idx)

        def drain(b):
            q = b & 1
            eoff = (s * nblk + b) * BLK
            pltpu.make_async_copy(src_hbm.at[pl.ds(eoff, BLK)], srcv.at[q],
                                  sem_idx).wait()
            pltpu.make_async_copy(dst_hbm.at[pl.ds(eoff, BLK)], dstv.at[q],
                                  sem_idx).wait()

        def flush(fl):
            parity = lax.rem(lax.div(fl, BLK), 2)
            for q in range(2):
                @pl.when(parity == q)
                def _(q=q):
                    pltpu.async_copy(u_hbm.at[sring.at[q]], rows,
                                     sem).wait()
                    pltpu.sync_copy(rows, acc_sh.at[lring.at[q]], add=True)

        for p in range(npass):
            lo = (c * npass + p) * csz

            def zrow(i, _):
                rows[lax.div(i, 8), pl.ds(lax.rem(i, 8) * 16, 16)] = (
                    jnp.zeros((16,), F32))
                return 0

            lax.fori_loop(0, BLK * 8, zrow, 0)
            off = 0
            while off < zrows:
                n = min(BLK, zrows - off)
                pltpu.sync_copy(rows.at[pl.ds(0, n)],
                                acc_sh.at[pl.ds(s * zrows + off, n)])
                off += n
            plsc.subcore_barrier()

            def blk_fn(b, carry):
                cnt, fl = carry
                q = b & 1
                drain(b)

                @pl.when(b + 1 < nblk)
                def _():
                    issue(b + 1)

                for g in range(BLK // 16):
                    d = dstv[q, pl.ds(g * 16, 16)]
                    sv = srcv[q, pl.ds(g * 16, 16)]
                    ld = d - lo
                    ok = (ld >= 0) & (ld < csz)
                    pos = cnt + jnp.cumsum(ok.astype(I32)) - 1
                    slot = pos & (2 * BLK - 1)
                    srow = slot >> 7
                    scol = slot & (BLK - 1)
                    plsc.store_scatter(sring, [srow, scol], sv, mask=ok)
                    plsc.store_scatter(lring, [srow, scol], ld, mask=ok)
                    cnt = cnt + jnp.sum(ok.astype(I32))

                @pl.when(cnt - fl >= BLK)
                def _():
                    flush(fl)

                fl = jnp.where(cnt - fl >= BLK, fl + BLK, fl)
                return cnt, fl

            issue(jnp.int32(0))
            cnt, fl = lax.fori_loop(0, nblk, blk_fn,
                                    (jnp.int32(0), jnp.int32(0)))

            for g in range(BLK // 16):
                slot = (cnt + g * 16 + lanes) & (2 * BLK - 1)
                srow = slot >> 7
                scol = slot & (BLK - 1)
                plsc.store_scatter(sring, [srow, scol],
                                   jnp.zeros((16,), I32))
                plsc.store_scatter(lring, [srow, scol],
                                   jnp.full((16,), csz, I32))
            flush(fl)
            plsc.subcore_barrier()

            pltpu.sync_copy(acc_sh.at[pl.ds(s * wb, wb)],
                            acc_hbm.at[pl.ds(lo + s * wb, wb)])
            if wrem:
                @pl.when(s == 0)
                def _():
                    pltpu.sync_copy(acc_sh.at[pl.ds(NS * wb, wrem)],
                                    acc_hbm.at[pl.ds(lo + NS * wb, wrem)])
            plsc.subcore_barrier()

    return edge_k


def _mm_first(z, w, cnt):
    mp, d = z.shape
    r = 1000

    def body(z_ref, w_ref, c_ref, o_ref):
        dis = lax.rsqrt(1.0 + c_ref[...][:, 0:1])
        o_ref[...] = jnp.dot(z_ref[...], w_ref[...],
                             preferred_element_type=F32) * dis

    return pl.pallas_call(
        body,
        grid=(mp // r,),
        in_specs=[
            pl.BlockSpec((r, d), lambda i: (i, 0)),
            pl.BlockSpec((d, d), lambda i: (0, 0)),
            pl.BlockSpec((r, 16), lambda i: (i, 0)),
        ],
        out_specs=pl.BlockSpec((r, d), lambda i: (i, 0)),
        out_shape=jax.ShapeDtypeStruct((mp, d), F32),
    )(z, w, cnt)


def _mm_mid(acc, u, cntp, bprev, w, cnt32):
    mp, d = acc.shape
    r = 1000

    def body(a_ref, u_ref, cp_ref, b_ref, w_ref, c32_ref, o_ref):
        disp = lax.rsqrt(1.0 + cp_ref[...][:, 0:1])
        x = jnp.maximum(disp * (a_ref[...] + u_ref[...]) + b_ref[...], 0.0)
        y = jnp.dot(x, w_ref[...], preferred_element_type=F32)
        c32 = c32_ref[...]
        d0 = lax.rsqrt(1.0 + c32[:, 0:1])
        d1 = lax.rsqrt(1.0 + c32[:, 16:17])
        o_ref[...] = jnp.concatenate([y * d0, y * d1], axis=1)

    return pl.pallas_call(
        body,
        grid=(mp // r,),
        in_specs=[
            pl.BlockSpec((r, d), lambda i: (i, 0)),
            pl.BlockSpec((r, d), lambda i: (i, 0)),
            pl.BlockSpec((r, 16), lambda i: (i, 0)),
            pl.BlockSpec((1, d), lambda i: (0, 0)),
            pl.BlockSpec((d, d), lambda i: (0, 0)),
            pl.BlockSpec((r, 32), lambda i: (i, 0)),
        ],
        out_specs=pl.BlockSpec((r, 2 * d), lambda i: (i, 0)),
        out_shape=jax.ShapeDtypeStruct((mp, 2 * d), F32),
    )(acc, u, cntp, bprev, w, cnt32)


def _fin_last(acc, u, cnt, b):
    m, d = acc.shape
    r = 2000

    def body(a_ref, u_ref, c_ref, b_ref, o_ref):
        dis = lax.rsqrt(1.0 + c_ref[...][:, 0:1])
        o_ref[...] = dis * (a_ref[...] + u_ref[...]) + b_ref[...]

    return pl.pallas_call(
        body,
        grid=(m // r,),
        in_specs=[
            pl.BlockSpec((r, d), lambda i: (i, 0)),
            pl.BlockSpec((r, d), lambda i: (i, 0)),
            pl.BlockSpec((r, 16), lambda i: (i, 0)),
            pl.BlockSpec((1, d), lambda i: (0, 0)),
        ],
        out_specs=pl.BlockSpec((r, d), lambda i: (i, 0)),
        out_shape=jax.ShapeDtypeStruct((m, d), F32),
    )(acc, u, cnt, b)


def _pad_edges(e):
    n = e.shape[1]
    p = (-n) % (NS * BLK)
    src = jnp.concatenate([e[0].astype(I32), jnp.zeros((p,), I32)])
    dst = jnp.concatenate([e[1].astype(I32), jnp.full((p,), -1, I32)])
    return src, dst


def kernel(z, edge_index, pool_edge2, pool_edge1, pool_edge0,
           W1, b1, W2, b2, W3, b3, W4, b4):
    n = z.shape[0]
    s1, d1 = _pad_edges(edge_index)
    s2, d2 = _pad_edges(pool_edge2)
    s3, d3 = _pad_edges(pool_edge1)
    s4, d4 = _pad_edges(pool_edge0)

    specs = [(n, s1.shape[0]), (2 * n, s2.shape[0]),
             (4 * n, s3.shape[0]), (8 * n, s4.shape[0])]
    cnt1, cnt2, cnt3, cnt4 = _make_count_kernel(specs)(d1, d2, d3, d4)

    u1 = _mm_first(z, W1, cnt1)
    acc1 = _make_edge_kernel(n, s1.shape[0], n // 2, 1)(u1, s1, d1)

    u2 = _mm_mid(acc1, u1, cnt1, b1.reshape(1, -1), W2,
                 cnt2.reshape(-1, 32)).reshape(2 * n, -1)
    acc2 = _make_edge_kernel(2 * n, s2.shape[0], n, 1)(u2, s2, d2)

    u3 = _mm_mid(acc2, u2, cnt2, b2.reshape(1, -1), W3,
                 cnt3.reshape(-1, 32)).reshape(4 * n, -1)
    acc3 = _make_edge_kernel(4 * n, s3.shape[0], n, 2)(u3, s3, d3)

    u4 = _mm_mid(acc3, u3, cnt3, b3.reshape(1, -1), W4,
                 cnt4.reshape(-1, 32)).reshape(8 * n, -1)
    acc4 = _make_edge_kernel(8 * n, s4.shape[0], n, 4)(u4, s4, d4)

    return _fin_last(acc4, u4, cnt4, b4.reshape(1, -1))
